# baseline (speedup 1.0000x reference)
import functools

import jax
import jax.numpy as jnp
from jax.experimental import pallas as pl
from jax.experimental.pallas import tpu as pltpu

E = 8
TOPK = 2
NEG = -1e30


def _routing(x, wr_p):
    T = x.shape[0]
    lg = jax.lax.dot_general(
        x, wr_p, (((1,), (1,)), ((), ())),
        preferred_element_type=jnp.float32,
    )
    lanes = jax.lax.broadcasted_iota(jnp.int32, (T, 128), 1)
    valid = lanes < E
    lg = jnp.where(valid, lg, NEG)
    m1 = jnp.max(lg, axis=1, keepdims=True)
    i1 = jnp.min(jnp.where(lg == m1, lanes, 999), axis=1, keepdims=True)
    lg2 = jnp.where(lanes == i1, NEG, lg)
    m2 = jnp.max(lg2, axis=1, keepdims=True)
    i2 = jnp.min(jnp.where(lg2 == m2, lanes, 999), axis=1, keepdims=True)
    d = jnp.exp(m2 - m1)
    w1 = 1.0 / (1.0 + d)
    w2 = 1.0 - w1
    comb = jnp.where(lanes == i1, w1, 0.0) + jnp.where(lanes == i2, w2, 0.0)
    return comb


def _silu(v):
    return v * (1.0 / (1.0 + jnp.exp(-v)))


_PREC = None


def _dot_t(a, b):
    return jax.lax.dot_general(a.astype(jnp.bfloat16), b.astype(jnp.bfloat16),
                               (((1,), (1,)), ((), ())),
                               preferred_element_type=jnp.float32,
                               precision=_PREC)


def _dense_body(x_ref, wr_ref, wg_ref, wu_ref, wd_ref, out_ref, comb_ref):
    e = pl.program_id(1)

    @pl.when(e == 0)
    def _():
        comb_ref[...] = _routing(x_ref[...], wr_ref[...])

    x = x_ref[...]
    wg = wg_ref[0]
    wu = wu_ref[0]
    wd = wd_ref[0]
    h = _silu(_dot_t(x, wg)) * _dot_t(x, wu)
    y = _dot_t(h, wd)
    lanes = jax.lax.broadcasted_iota(jnp.int32, comb_ref.shape, 1)
    wcol = jnp.sum(jnp.where(lanes == e, comb_ref[...], 0.0), axis=1,
                   keepdims=True)

    @pl.when(e == 0)
    def _():
        out_ref[...] = wcol * y

    @pl.when(e > 0)
    def _():
        out_ref[...] += wcol * y


def kernel(hidden_states, W_router, W_gate, W_up, W_down):
    b, s, d = hidden_states.shape
    T = b * s
    x = hidden_states.reshape(T, d)
    wr_p = jnp.zeros((128, d), jnp.float32).at[:E].set(W_router)

    TB = 512
    out = pl.pallas_call(
        _dense_body,
        grid=(T // TB, E),
        in_specs=[
            pl.BlockSpec((TB, d), lambda t, e: (t, 0)),
            pl.BlockSpec((128, d), lambda t, e: (0, 0)),
            pl.BlockSpec((1, W_gate.shape[1], d), lambda t, e: (e, 0, 0)),
            pl.BlockSpec((1, W_up.shape[1], d), lambda t, e: (e, 0, 0)),
            pl.BlockSpec((1, d, W_down.shape[2]), lambda t, e: (e, 0, 0)),
        ],
        out_specs=pl.BlockSpec((TB, d), lambda t, e: (t, 0)),
        out_shape=jax.ShapeDtypeStruct((T, d), jnp.float32),
        scratch_shapes=[pltpu.VMEM((TB, 128), jnp.float32)],
    )(x, wr_p, W_gate, W_up, W_down)
    return out.reshape(b, s, d)


# speedup vs baseline: 1.2276x; 1.2276x over previous
import functools

import jax
import jax.numpy as jnp
from jax.experimental import pallas as pl
from jax.experimental.pallas import tpu as pltpu

E = 8
TOPK = 2
NEG = -1e30


def _routing(x, wr_p):
    T = x.shape[0]
    lg = jax.lax.dot_general(
        x, wr_p, (((1,), (1,)), ((), ())),
        preferred_element_type=jnp.float32,
    )
    lanes = jax.lax.broadcasted_iota(jnp.int32, (T, 128), 1)
    valid = lanes < E
    lg = jnp.where(valid, lg, NEG)
    m1 = jnp.max(lg, axis=1, keepdims=True)
    i1 = jnp.min(jnp.where(lg == m1, lanes, 999), axis=1, keepdims=True)
    lg2 = jnp.where(lanes == i1, NEG, lg)
    m2 = jnp.max(lg2, axis=1, keepdims=True)
    i2 = jnp.min(jnp.where(lg2 == m2, lanes, 999), axis=1, keepdims=True)
    d = jnp.exp(m2 - m1)
    w1 = 1.0 / (1.0 + d)
    w2 = 1.0 - w1
    comb = jnp.where(lanes == i1, w1, 0.0) + jnp.where(lanes == i2, w2, 0.0)
    return comb


def _silu(v):
    return v * (1.0 / (1.0 + jnp.exp(-v)))


_PREC = None


def _dot_t(a, b):
    return jax.lax.dot_general(a.astype(jnp.bfloat16), b.astype(jnp.bfloat16),
                               (((1,), (1,)), ((), ())),
                               preferred_element_type=jnp.float32,
                               precision=_PREC)


def _dense_body(x_ref, wr_ref, wg_ref, wu_ref, wd_ref, out_ref, comb_ref):
    e = pl.program_id(1)

    @pl.when(e == 0)
    def _():
        comb_ref[...] = _routing(x_ref[...], wr_ref[...])

    x = x_ref[...]
    wg = wg_ref[0]
    wu = wu_ref[0]
    wd = wd_ref[0]
    h = _silu(_dot_t(x, wg)) * _dot_t(x, wu)
    y = _dot_t(h, wd)
    lanes = jax.lax.broadcasted_iota(jnp.int32, comb_ref.shape, 1)
    wcol = jnp.sum(jnp.where(lanes == e, comb_ref[...], 0.0), axis=1,
                   keepdims=True)

    @pl.when(e == 0)
    def _():
        out_ref[...] = wcol * y

    @pl.when(e > 0)
    def _():
        out_ref[...] += wcol * y


def kernel(hidden_states, W_router, W_gate, W_up, W_down):
    b, s, d = hidden_states.shape
    T = b * s
    x = hidden_states.reshape(T, d)
    wr_p = jnp.zeros((128, d), jnp.float32).at[:E].set(W_router)

    TB = 2048
    out = pl.pallas_call(
        _dense_body,
        grid=(T // TB, E),
        in_specs=[
            pl.BlockSpec((TB, d), lambda t, e: (t, 0)),
            pl.BlockSpec((128, d), lambda t, e: (0, 0)),
            pl.BlockSpec((1, W_gate.shape[1], d), lambda t, e: (e, 0, 0)),
            pl.BlockSpec((1, W_up.shape[1], d), lambda t, e: (e, 0, 0)),
            pl.BlockSpec((1, d, W_down.shape[2]), lambda t, e: (e, 0, 0)),
        ],
        out_specs=pl.BlockSpec((TB, d), lambda t, e: (t, 0)),
        out_shape=jax.ShapeDtypeStruct((T, d), jnp.float32),
        scratch_shapes=[pltpu.VMEM((TB, 128), jnp.float32)],
    )(x, wr_p, W_gate, W_up, W_down)
    return out.reshape(b, s, d)
